# Initial kernel scaffold; baseline (speedup 1.0000x reference)
import jax
import jax.numpy as jnp
from jax.experimental import pallas as pl


def kernel(x_SB, x_PV, x_PQ, x_NB, edge_index_SB_PV, edge_index_SB_PQ, edge_index_SB_NB, edge_index_PV_PQ, edge_index_PV_NB, edge_index_PV_PV, edge_index_PQ_NB, edge_index_PQ_PQ, edge_index_NB_NB, edge_attr_SB_PV, edge_attr_SB_PQ, edge_attr_SB_NB, edge_attr_PV_PQ, edge_attr_PV_NB, edge_attr_PV_PV, edge_attr_PQ_NB, edge_attr_PQ_PQ, edge_attr_NB_NB, params):
    raise NotImplementedError("write your pallas kernel here")


# speedup vs baseline: 1.9615x; 1.9615x over previous
import functools
import math

import jax
import jax.numpy as jnp
from jax import lax
from jax.experimental import pallas as pl
from jax.experimental.pallas import tpu as pltpu

_NODE_TYPES = ['SB', 'PV', 'PQ', 'NB']
_EDGE_TYPES = [('SB', 'PV'), ('SB', 'PQ'), ('SB', 'NB'), ('PV', 'PQ'),
               ('PV', 'NB'), ('PV', 'PV'), ('PQ', 'NB'), ('PQ', 'PQ'),
               ('NB', 'NB')]
_N = 2500
_E = 35000
_D = 128
_B = 500
_NBLK = _E // _B
_NT = len(_EDGE_TYPES)
_F32 = jnp.float32


def _conv_body(src_ref, dst_ref, ea_ref, xsrc_ref, xdst_ref,
               wq_ref, bq_ref, wk_ref, bk_ref, wv_ref, bv_ref,
               we_ref, be_ref, wskip_ref, bskip_ref,
               out_ref, q_s, k_s, v_s, acc_s, den_s):
    i = pl.program_id(1)

    @pl.when(i == 0)
    def _init():
        xs = xsrc_ref[0]
        xd = xdst_ref[0]
        q_s[...] = jnp.dot(xd, wq_ref[0], preferred_element_type=_F32) + bq_ref[0]
        k_s[...] = jnp.dot(xs, wk_ref[0], preferred_element_type=_F32) + bk_ref[0]
        v_s[...] = jnp.dot(xs, wv_ref[0], preferred_element_type=_F32) + bv_ref[0]
        acc_s[...] = jnp.zeros_like(acc_s)
        den_s[...] = jnp.zeros_like(den_s)

    src = src_ref[0]
    dst = dst_ref[0]
    iota = lax.broadcasted_iota(jnp.int32, (_N, _B), 0)
    oh_s = (iota == jnp.broadcast_to(src, (_N, _B))).astype(_F32)
    oh_d = (iota == jnp.broadcast_to(dst, (_N, _B))).astype(_F32)

    dn_t = (((0,), (0,)), ((), ()))
    dn_s = (((1,), (0,)), ((), ()))
    kb = lax.dot_general(oh_s, k_s[...], dn_t, preferred_element_type=_F32)
    vb = lax.dot_general(oh_s, v_s[...], dn_t, preferred_element_type=_F32)
    qb = lax.dot_general(oh_d, q_s[...], dn_t, preferred_element_type=_F32)
    eb = jnp.dot(ea_ref[0], we_ref[0], preferred_element_type=_F32) + be_ref[0]

    inv_sqrt_d = 1.0 / math.sqrt(_D)
    logits = jnp.sum(qb * (kb + eb), axis=1, keepdims=True) * inv_sqrt_d
    ex = jnp.exp(logits)
    contrib = ex * (vb + eb)
    acc_s[...] += lax.dot_general(oh_d, contrib, dn_s, preferred_element_type=_F32)
    den_s[...] += lax.dot_general(oh_d, jnp.broadcast_to(ex, (_B, 8)), dn_s,
                                  preferred_element_type=_F32)

    @pl.when(i == _NBLK - 1)
    def _fin():
        skip = jnp.dot(xdst_ref[0], wskip_ref[0],
                       preferred_element_type=_F32) + bskip_ref[0]
        out_ref[0] = acc_s[...] / (den_s[:, 0:1] + 1e-16) + skip


@jax.jit
def _hetero_layer(src_all, dst_all, ea_all, xsrc_all, xdst_all,
                  wq, bq, wk, bk, wv, bv, we, be, wsk, bsk):
    full_nd = pl.BlockSpec((1, _N, _D), lambda t, i: (t, 0, 0))
    full_dd = pl.BlockSpec((1, _D, _D), lambda t, i: (t, 0, 0))
    full_b = pl.BlockSpec((1, 1, _D), lambda t, i: (t, 0, 0))
    idx_spec = pl.BlockSpec((1, 1, _B), lambda t, i: (t * _NBLK + i, 0, 0))
    ea_spec = pl.BlockSpec((1, _B, 2), lambda t, i: (t * _NBLK + i, 0, 0))
    return pl.pallas_call(
        _conv_body,
        grid=(_NT, _NBLK),
        in_specs=[idx_spec, idx_spec, ea_spec, full_nd, full_nd,
                  full_dd, full_b, full_dd, full_b, full_dd, full_b,
                  pl.BlockSpec((1, 2, _D), lambda t, i: (t, 0, 0)), full_b,
                  full_dd, full_b],
        out_specs=pl.BlockSpec((1, _N, _D), lambda t, i: (t, 0, 0)),
        out_shape=jax.ShapeDtypeStruct((_NT, _N, _D), _F32),
        scratch_shapes=[pltpu.VMEM((_N, _D), _F32)] * 4 + [pltpu.VMEM((_N, 8), _F32)],
        compiler_params=pltpu.CompilerParams(
            dimension_semantics=("arbitrary", "arbitrary")),
    )(src_all, dst_all, ea_all, xsrc_all, xdst_all,
      wq, bq, wk, bk, wv, bv, we, be, wsk, bsk)


def _linear_body(x_ref, w_ref, b_ref, o_ref):
    o_ref[...] = jnp.dot(x_ref[...], w_ref[...],
                         preferred_element_type=_F32) + b_ref[...]


@jax.jit
def _final_linear(x, w, b):
    return pl.pallas_call(
        _linear_body,
        out_shape=jax.ShapeDtypeStruct((_N, _D), _F32),
    )(x, w, b.reshape(1, _D))


def kernel(x_SB, x_PV, x_PQ, x_NB,
           edge_index_SB_PV, edge_index_SB_PQ, edge_index_SB_NB,
           edge_index_PV_PQ, edge_index_PV_NB, edge_index_PV_PV,
           edge_index_PQ_NB, edge_index_PQ_PQ, edge_index_NB_NB,
           edge_attr_SB_PV, edge_attr_SB_PQ, edge_attr_SB_NB,
           edge_attr_PV_PQ, edge_attr_PV_NB, edge_attr_PV_PV,
           edge_attr_PQ_NB, edge_attr_PQ_PQ, edge_attr_NB_NB,
           params):
    loc = dict(locals())
    eis = {'%s_%s' % (s, d): loc['edge_index_%s_%s' % (s, d)]
           for (s, d) in _EDGE_TYPES}
    eas = {'%s_%s' % (s, d): loc['edge_attr_%s_%s' % (s, d)]
           for (s, d) in _EDGE_TYPES}

    src_all = jnp.concatenate(
        [eis['%s_%s' % (s, d)][0].reshape(_NBLK, 1, _B)
         for (s, d) in _EDGE_TYPES], axis=0)
    dst_all = jnp.concatenate(
        [eis['%s_%s' % (s, d)][1].reshape(_NBLK, 1, _B)
         for (s, d) in _EDGE_TYPES], axis=0)
    ea_all = jnp.concatenate(
        [eas['%s_%s' % (s, d)].reshape(_NBLK, _B, 2)
         for (s, d) in _EDGE_TYPES], axis=0)

    x = {'SB': x_SB, 'PV': x_PV, 'PQ': x_PQ, 'NB': x_NB}
    for layer in params['convs']:
        xsrc_all = jnp.stack([x[s] for (s, d) in _EDGE_TYPES])
        xdst_all = jnp.stack([x[d] for (s, d) in _EDGE_TYPES])
        pk = ['%s_%s' % (s, d) for (s, d) in _EDGE_TYPES]
        wq = jnp.stack([layer[k]['Wq'] for k in pk])
        bq = jnp.stack([layer[k]['bq'] for k in pk]).reshape(_NT, 1, _D)
        wk = jnp.stack([layer[k]['Wk'] for k in pk])
        bk = jnp.stack([layer[k]['bk'] for k in pk]).reshape(_NT, 1, _D)
        wv = jnp.stack([layer[k]['Wv'] for k in pk])
        bv = jnp.stack([layer[k]['bv'] for k in pk]).reshape(_NT, 1, _D)
        we = jnp.stack([layer[k]['We'] for k in pk])
        be = jnp.stack([layer[k]['be'] for k in pk]).reshape(_NT, 1, _D)
        wsk = jnp.stack([layer[k]['Wskip'] for k in pk])
        bsk = jnp.stack([layer[k]['bskip'] for k in pk]).reshape(_NT, 1, _D)

        conv = _hetero_layer(src_all, dst_all, ea_all, xsrc_all, xdst_all,
                             wq, bq, wk, bk, wv, bv, we, be, wsk, bsk)

        agg = {}
        for ti, (s, d) in enumerate(_EDGE_TYPES):
            agg[d] = agg[d] + conv[ti] if d in agg else conv[ti]
        x = {nt: (jax.nn.relu(agg[nt]) if nt in agg else x[nt])
             for nt in _NODE_TYPES}

    return _final_linear(x['NB'], params['lin_w'], params['lin_b'])
